# Initial kernel scaffold; baseline (speedup 1.0000x reference)
import jax
import jax.numpy as jnp
from jax.experimental import pallas as pl


def kernel(user_table, sem_tables, W1, b1, W2, b2, W3, b3, user, sem_codes):
    raise NotImplementedError("write your pallas kernel here")


# speedup vs baseline: 1.0865x; 1.0865x over previous
import functools

import jax
import jax.numpy as jnp
import numpy as np
from jax import lax
from jax.experimental import pallas as pl
from jax.experimental.pallas import tpu as pltpu
from jax.experimental.pallas import tpu_sc as plsc

B = 16384
D = 16
SEM_CODEBOOK = 256
SEM_LEVELS = 4
NUM_CAT = 1 + SEM_LEVELS
_IU = np.triu_indices(NUM_CAT, k=1)
PAIR_N = [int(x) for x in _IU[0]]
PAIR_M = [int(x) for x in _IU[1]]
NPAIR = len(PAIR_N)

BBLK = 2048


@functools.cache
def _make_gather():
    info = plsc.get_sparse_core_info()
    NC, NS = info.num_cores, info.num_subcores
    NW = NC * NS
    b_per_w = B // NW
    mesh = plsc.VectorSubcoreMesh(core_axis_name="c", subcore_axis_name="s")

    @functools.partial(
        pl.kernel,
        out_type=jax.ShapeDtypeStruct((NUM_CAT, B, D), jnp.float32),
        mesh=mesh,
        scratch_types=[
            pltpu.VMEM((b_per_w,), jnp.int32),
            pltpu.VMEM((b_per_w, D), jnp.float32),
            pltpu.SemaphoreType.DMA,
        ],
        compiler_params=pltpu.CompilerParams(use_tc_tiling_on_sc=False),
    )
    def gather_kernel(user_table, sem_flat, user_idx, sem_codes_t, out,
                      idx_v, rows_v, sem):
        wid = lax.axis_index("s") * NC + lax.axis_index("c")
        base = wid * b_per_w
        pltpu.sync_copy(user_idx.at[pl.ds(base, b_per_w)], idx_v)
        pltpu.async_copy(user_table.at[idx_v], rows_v, sem).wait()
        pltpu.sync_copy(rows_v, out.at[0, pl.ds(base, b_per_w)])
        for l in range(SEM_LEVELS):
            pltpu.sync_copy(sem_codes_t.at[l, pl.ds(base, b_per_w)], idx_v)
            for j in range(b_per_w // 16):
                v = idx_v[pl.ds(j * 16, 16)]
                v = jnp.clip(v, 0, SEM_CODEBOOK - 1) + l * SEM_CODEBOOK
                idx_v[pl.ds(j * 16, 16)] = v
            pltpu.async_copy(sem_flat.at[idx_v], rows_v, sem).wait()
            pltpu.sync_copy(rows_v, out.at[l + 1, pl.ds(base, b_per_w)])

    return gather_kernel


def _mlp_body(feats, w1e, b1, w2, b2, w3t, b3, out):
    f = feats[...]
    a = jnp.concatenate([f[n] for n in PAIR_N], axis=1)
    bm = jnp.concatenate([f[m] for m in PAIR_M], axis=1)
    p = a * bm
    h = jnp.dot(p, w1e[...], preferred_element_type=jnp.float32) + b1[...]
    h = jnp.maximum(h, 0.0)
    h = jnp.dot(h, w2[...], preferred_element_type=jnp.float32) + b2[...]
    h = jnp.maximum(h, 0.0)
    z = jnp.sum(h * w3t[...], axis=1, keepdims=True) + b3[...]
    out[...] = 1.0 / (1.0 + jnp.exp(-z))


def _run_mlp(feats, W1e, b1, W2, b2, W3t, b3):
    return pl.pallas_call(
        _mlp_body,
        grid=(B // BBLK,),
        in_specs=[
            pl.BlockSpec((NUM_CAT, BBLK, D), lambda i: (0, i, 0)),
            pl.BlockSpec((D * NPAIR, 128), lambda i: (0, 0)),
            pl.BlockSpec((1, 128), lambda i: (0, 0)),
            pl.BlockSpec((128, 64), lambda i: (0, 0)),
            pl.BlockSpec((1, 64), lambda i: (0, 0)),
            pl.BlockSpec((1, 64), lambda i: (0, 0)),
            pl.BlockSpec((1, 1), lambda i: (0, 0)),
        ],
        out_specs=pl.BlockSpec((BBLK, 1), lambda i: (i, 0)),
        out_shape=jax.ShapeDtypeStruct((B, 1), jnp.float32),
    )(feats, W1e, b1, W2, b2, W3t, b3)


def kernel(user_table, sem_tables, W1, b1, W2, b2, W3, b3, user, sem_codes):
    sem_flat = sem_tables.reshape(SEM_LEVELS * SEM_CODEBOOK, D)
    user_idx = user.astype(jnp.int32)
    sem_codes_t = sem_codes.astype(jnp.int32).T
    feats = _make_gather()(user_table, sem_flat, user_idx, sem_codes_t)
    W1e = jnp.repeat(W1, D, axis=0)
    out = _run_mlp(feats, W1e, b1.reshape(1, -1), W2, b2.reshape(1, -1),
                   W3.reshape(1, -1), b3.reshape(1, 1))
    return out.reshape(-1)
